# Initial kernel scaffold; baseline (speedup 1.0000x reference)
import jax
import jax.numpy as jnp
from jax.experimental import pallas as pl


def kernel(data, spatial_weight, temporal_weight):
    raise NotImplementedError("write your pallas kernel here")


# speedup vs baseline: 36.1850x; 36.1850x over previous
import jax
import jax.numpy as jnp
from jax.experimental import pallas as pl

_DIM = 4096
_LEVELS = 256
_B = 4
_T = 8
_HW = 1024
_BT = _B * _T
_CHUNK = 128


def _encode_kernel(flat_ref, sw_ref, tw_ref, out_ref):
    flat = flat_ref[...]
    mx = jnp.max(flat, axis=1, keepdims=True)
    scaled = flat / mx * (_LEVELS - 1.0)
    q = jnp.clip(jnp.round(scaled), 0.0, _LEVELS - 1.0).astype(jnp.int32)

    levels = jax.lax.broadcasted_iota(jnp.int32, (1, 1, _LEVELS), 2)

    counts = jnp.zeros((_BT, _LEVELS), jnp.float32)
    for i in range(_HW // _CHUNK):
        chunk = q[:, i * _CHUNK:(i + 1) * _CHUNK]
        onehot = (chunk[:, :, None] == levels).astype(jnp.float32)
        counts = counts + jnp.sum(onehot, axis=1)

    w = jax.lax.dot_general(
        counts, sw_ref[...],
        (((1,), (0,)), ((), ())),
        preferred_element_type=jnp.float32,
        precision=jax.lax.Precision.HIGHEST)

    hv = jnp.zeros((_B, _DIM), jnp.float32)
    for t in range(_T):
        hv = hv + w[t * _B:(t + 1) * _B, :] * tw_ref[t:t + 1, :]
    out_ref[...] = jnp.sign(hv)


def kernel(data, spatial_weight, temporal_weight):
    b, t, c, h, w = data.shape
    flat = jnp.transpose(data.reshape(b, t, c * h * w), (1, 0, 2))
    flat = flat.reshape(t * b, c * h * w)
    return pl.pallas_call(
        _encode_kernel,
        out_shape=jax.ShapeDtypeStruct((b, _DIM), jnp.float32),
    )(flat, spatial_weight[:_LEVELS], temporal_weight[:t])


# speedup vs baseline: 55.1811x; 1.5250x over previous
import jax
import jax.numpy as jnp
from jax.experimental import pallas as pl

_DIM = 4096
_LEVELS = 256
_B = 4
_T = 8
_HW = 1024
_BT = _B * _T
_CHUNK = 128


def _encode_kernel(flat_ref, sw_ref, tw_ref, out_ref):
    flat = flat_ref[...]
    mx = jnp.max(flat, axis=1, keepdims=True)
    scaled = flat / mx * (_LEVELS - 1.0)
    q = jnp.clip(jnp.round(scaled), 0.0, _LEVELS - 1.0).astype(jnp.int32)

    levels = jax.lax.broadcasted_iota(jnp.int32, (1, 1, _LEVELS), 2)

    counts = jnp.zeros((_BT, _LEVELS), jnp.float32)
    for i in range(_HW // _CHUNK):
        chunk = q[:, i * _CHUNK:(i + 1) * _CHUNK]
        onehot = (chunk[:, :, None] == levels).astype(jnp.float32)
        counts = counts + jnp.sum(onehot, axis=1)

    w = jax.lax.dot_general(
        counts, sw_ref[...],
        (((1,), (0,)), ((), ())),
        preferred_element_type=jnp.float32,
        precision=jax.lax.Precision.HIGHEST)

    tw = tw_ref[...]
    m = w * jnp.concatenate([tw] * _B, axis=0)
    rows = [jnp.sum(m[b * _T:(b + 1) * _T, :], axis=0, keepdims=True)
            for b in range(_B)]
    out_ref[...] = jnp.sign(jnp.concatenate(rows, axis=0))


def kernel(data, spatial_weight, temporal_weight):
    b, t, c, h, w = data.shape
    flat = data.reshape(b * t, c * h * w)
    return pl.pallas_call(
        _encode_kernel,
        grid=(1,),
        out_shape=jax.ShapeDtypeStruct((b, _DIM), jnp.float32),
        in_specs=[
            pl.BlockSpec((_BT, _HW), lambda i: (0, 0)),
            pl.BlockSpec((_LEVELS, _DIM), lambda i: (0, 0)),
            pl.BlockSpec((t, _DIM), lambda i: (0, 0)),
        ],
        out_specs=pl.BlockSpec((b, _DIM), lambda i: (0, 0)),
    )(flat, spatial_weight, temporal_weight)


# speedup vs baseline: 101.1038x; 1.8322x over previous
import jax
import jax.numpy as jnp
from jax.experimental import pallas as pl

_DIM = 4096
_LEVELS = 256
_B = 4
_T = 8
_HW = 1024
_BT = _B * _T
_CHUNK = 128


def _encode_kernel(flat_ref, sw_ref, tw_ref, out_ref):
    flat = flat_ref[...]
    mx = jnp.max(flat, axis=1, keepdims=True)
    scaled = flat / mx * (_LEVELS - 1.0)
    q = jnp.clip(jnp.round(scaled), 0.0, _LEVELS - 1.0).astype(jnp.int32)

    nib = jax.lax.broadcasted_iota(jnp.int32, (1, 16, 1), 1)
    q3 = q[:, None, :]
    hi = ((q3 >> 4) == nib).astype(jnp.bfloat16)
    lo = ((q3 & 15) == nib).astype(jnp.bfloat16)
    counts3 = jax.lax.dot_general(
        hi, lo,
        (((2,), (2,)), ((0,), (0,))),
        preferred_element_type=jnp.float32)
    counts = counts3.reshape(_BT, _LEVELS)

    c_hi = jnp.floor(counts * (1.0 / 16.0))
    c_lo = counts - c_hi * 16.0
    sw = sw_ref[...].astype(jnp.bfloat16)
    dot = lambda a: jax.lax.dot_general(
        a.astype(jnp.bfloat16), sw,
        (((1,), (0,)), ((), ())),
        preferred_element_type=jnp.float32)
    w = dot(c_hi) * 16.0 + dot(c_lo)

    tw = tw_ref[...]
    m = w * jnp.concatenate([tw] * _B, axis=0)
    rows = [jnp.sum(m[b * _T:(b + 1) * _T, :], axis=0, keepdims=True)
            for b in range(_B)]
    out_ref[...] = jnp.sign(jnp.concatenate(rows, axis=0))


def kernel(data, spatial_weight, temporal_weight):
    b, t, c, h, w = data.shape
    flat = data.reshape(b * t, c * h * w)
    return pl.pallas_call(
        _encode_kernel,
        grid=(1,),
        out_shape=jax.ShapeDtypeStruct((b, _DIM), jnp.float32),
        in_specs=[
            pl.BlockSpec((_BT, _HW), lambda i: (0, 0)),
            pl.BlockSpec((_LEVELS, _DIM), lambda i: (0, 0)),
            pl.BlockSpec((t, _DIM), lambda i: (0, 0)),
        ],
        out_specs=pl.BlockSpec((b, _DIM), lambda i: (0, 0)),
    )(flat, spatial_weight, temporal_weight)
